# baseline (speedup 1.0000x reference)
import functools

import jax
import jax.numpy as jnp
from jax import lax
from jax.experimental import pallas as pl
from jax.experimental.pallas import tpu as pltpu
from jax.experimental.pallas import tpu_sc as plsc

MEMORY_DIM = 32
TIME_DIM = 16
EDGE_FEAT_DIM = 16
HIDDEN = 128

_NC = 2
_NS = 16
_NW = _NC * _NS
_CHUNK = 128
_LANES = 16
_QSH = 18
_VP = 1 << _QSH
_RB = 8192


def _tc_relayout(memT):
    grid = (_VP // _RB,)
    last_blk = (memT.shape[1] + _RB - 1) // _RB - 1

    def body(q0, q1, q2, q3, outb):
        qcat = jnp.concatenate([q0[:], q1[:], q2[:], q3[:]], axis=0)
        eye = jnp.eye(128, dtype=jnp.float32)
        outb[:] = lax.dot_general(
            qcat, eye, (((0,), (0,)), ((), ())),
            preferred_element_type=jnp.float32)

    qspec = lambda k: pl.BlockSpec(
        (MEMORY_DIM, _RB),
        lambda i, k=k: (0, jnp.minimum(k * (_VP // _RB) + i, last_blk)))
    return pl.pallas_call(
        body,
        grid=grid,
        in_specs=[qspec(0), qspec(1), qspec(2), qspec(3)],
        out_specs=pl.BlockSpec((_RB, 128), lambda i: (i, 0)),
        out_shape=jax.ShapeDtypeStruct((_VP, 128), jnp.float32),
    )(memT, memT, memT, memT)


def _sc_gather_extract(mem128, src, dst, t, last_update):
    B = src.shape[0]
    b_per_w = B // _NW
    hb_rows = b_per_w // 2
    mesh = plsc.VectorSubcoreMesh(core_axis_name="c", subcore_axis_name="s")

    @functools.partial(
        pl.kernel,
        mesh=mesh,
        compiler_params=pltpu.CompilerParams(needs_layout_passes=False),
        out_type=jax.ShapeDtypeStruct((B, 128), jnp.float32),
        scratch_types=[
            pltpu.VMEM((b_per_w,), jnp.int32),
            pltpu.VMEM((b_per_w,), jnp.int32),
            pltpu.VMEM((b_per_w,), jnp.int32),
            pltpu.VMEM((b_per_w,), jnp.int32),
            pltpu.VMEM((b_per_w,), jnp.int32),
            pltpu.VMEM((b_per_w,), jnp.int32),
            pltpu.VMEM((hb_rows, 128), jnp.float32),
            pltpu.VMEM((hb_rows, 128), jnp.float32),
            pltpu.VMEM((hb_rows, 128), jnp.float32),
            pltpu.VMEM((b_per_w,), jnp.float32),
            pltpu.VMEM((b_per_w,), jnp.float32),
            pltpu.VMEM((b_per_w,), jnp.float32),
            pltpu.SemaphoreType.DMA,
        ],
    )
    def k(mem_hbm, src_hbm, dst_hbm, t_hbm, lu_hbm, feat,
          sidx, didx, sp, dp, scb, dcb, srows, drows, fbuf, slu, tv, dtv, sem):
        wid = lax.axis_index("s") * _NC + lax.axis_index("c")
        base = wid * b_per_w
        pltpu.sync_copy(src_hbm.at[pl.ds(base, b_per_w)], sidx)
        pltpu.sync_copy(dst_hbm.at[pl.ds(base, b_per_w)], didx)
        pltpu.sync_copy(t_hbm.at[pl.ds(base, b_per_w)], tv)
        for i in range(b_per_w // _LANES):
            s = pl.ds(i * _LANES, _LANES)
            r = sidx[s]
            sp[s] = jnp.bitwise_and(r, _VP - 1)
            scb[s] = lax.shift_right_logical(r, _QSH) * 32
            r2 = didx[s]
            dp[s] = jnp.bitwise_and(r2, _VP - 1)
            dcb[s] = lax.shift_right_logical(r2, _QSH) * 32
        lus = [pltpu.async_copy(lu_hbm.at[sidx.at[pl.ds(j * _CHUNK, _CHUNK)]],
                                slu.at[pl.ds(j * _CHUNK, _CHUNK)], sem)
               for j in range(b_per_w // _CHUNK)]
        for c in lus:
            c.wait()
        for i in range(b_per_w // _LANES):
            s = pl.ds(i * _LANES, _LANES)
            dtv[s] = tv[s] - slu[s]

        lane = lax.iota(jnp.int32, _LANES)
        for h in range(2):
            hb = h * hb_rows
            cs = []
            for j in range(hb_rows // _CHUNK):
                sl = pl.ds(hb + j * _CHUNK, _CHUNK)
                dsl = pl.ds(j * _CHUNK, _CHUNK)
                cs.append(pltpu.async_copy(mem_hbm.at[sp.at[sl]], srows.at[dsl], sem))
                cs.append(pltpu.async_copy(mem_hbm.at[dp.at[sl]], drows.at[dsl], sem))
            for c in cs:
                c.wait()

            def ebody(g, carry):
                rows = lane + g * _LANES
                scv = plsc.load_gather(scb, [rows + hb])
                dcv = plsc.load_gather(dcb, [rows + hb])
                dtl = plsc.load_gather(dtv, [rows + hb])
                for j in range(MEMORY_DIM):
                    v = plsc.load_gather(srows, [rows, scv + j])
                    plsc.store_scatter(fbuf, [rows, jnp.full((_LANES,), j, jnp.int32)], v)
                    v2 = plsc.load_gather(drows, [rows, dcv + j])
                    plsc.store_scatter(
                        fbuf, [rows, jnp.full((_LANES,), MEMORY_DIM + j, jnp.int32)], v2)
                plsc.store_scatter(
                    fbuf, [rows, jnp.full((_LANES,), 2 * MEMORY_DIM, jnp.int32)], dtl)
                return carry

            lax.fori_loop(0, hb_rows // _LANES, ebody, 0)
            pltpu.sync_copy(fbuf, feat.at[pl.ds(base + hb, hb_rows)])

    return k(mem128, src, dst, t, last_update)


def _mlp_body(fb, eat, wt, bt, w1a, w1b, w1c, w1d, b1r, w2t, b2r, outT):
    sm = fb[:, 0:MEMORY_DIM]
    dm = fb[:, MEMORY_DIM:2 * MEMORY_DIM]
    dtc = fb[:, 2 * MEMORY_DIM:2 * MEMORY_DIM + 1]
    enc = jnp.cos(dtc * wt[:] + bt[:])
    h = jnp.dot(sm, w1a[:], preferred_element_type=jnp.float32)
    h += jnp.dot(dm, w1b[:], preferred_element_type=jnp.float32)
    h += jnp.dot(enc, w1c[:], preferred_element_type=jnp.float32)
    h += lax.dot_general(eat[:], w1d[:], (((0,), (0,)), ((), ())),
                         preferred_element_type=jnp.float32)
    h = jnp.maximum(h + b1r[:], 0.0)
    outT[:] = lax.dot_general(w2t[:], h, (((1,), (1,)), ((), ())),
                              preferred_element_type=jnp.float32) + b2r[0, 0]


def _tc_mlp(feat, edge_attr_T, W_time, b_time, W1, b1, W2_T, b2):
    B = feat.shape[0]
    BLK = 8192
    grid = (B // BLK,)
    blk = lambda r, c: pl.BlockSpec((r, c), lambda i: (i, 0))
    full = lambda r, c: pl.BlockSpec((r, c), lambda i: (0, 0))
    outT = pl.pallas_call(
        _mlp_body,
        grid=grid,
        in_specs=[
            blk(BLK, 128),
            pl.BlockSpec((EDGE_FEAT_DIM, BLK), lambda i: (0, i)),
            full(1, TIME_DIM),
            full(1, TIME_DIM),
            full(MEMORY_DIM, HIDDEN),
            full(MEMORY_DIM, HIDDEN),
            full(TIME_DIM, HIDDEN),
            full(EDGE_FEAT_DIM, HIDDEN),
            full(1, HIDDEN),
            full(1, HIDDEN),
            full(1, 1),
        ],
        out_specs=pl.BlockSpec((1, BLK), lambda i: (0, i)),
        out_shape=jax.ShapeDtypeStruct((1, B), jnp.float32),
    )(feat, edge_attr_T, W_time, b_time,
      W1[0:MEMORY_DIM], W1[MEMORY_DIM:2 * MEMORY_DIM],
      W1[2 * MEMORY_DIM:2 * MEMORY_DIM + TIME_DIM],
      W1[2 * MEMORY_DIM + TIME_DIM:],
      b1.reshape(1, HIDDEN), W2_T, b2.reshape(1, 1))
    return outT.T


def kernel(src, dst, t, edge_attr, memory, last_update,
           W_time, b_time, W1, b1, W2, b2):
    mem128 = _tc_relayout(memory.T)
    feat = _sc_gather_extract(
        mem128, src.astype(jnp.int32), dst.astype(jnp.int32), t, last_update)
    return _tc_mlp(feat, edge_attr.astype(jnp.float32).T,
                   W_time, b_time.reshape(1, TIME_DIM), W1, b1, W2.T, b2)


# speedup vs baseline: 1.0060x; 1.0060x over previous
import functools

import jax
import jax.numpy as jnp
from jax import lax
from jax.experimental import pallas as pl
from jax.experimental.pallas import tpu as pltpu
from jax.experimental.pallas import tpu_sc as plsc

MEMORY_DIM = 32
TIME_DIM = 16
EDGE_FEAT_DIM = 16
HIDDEN = 128

_NC = 2
_NS = 16
_NW = _NC * _NS
_CHUNK = 128
_LANES = 16
_QSH = 18
_VP = 1 << _QSH
_RB = 8192


def _tc_relayout(memT):
    grid = (_VP // _RB,)
    last_blk = (memT.shape[1] + _RB - 1) // _RB - 1

    def body(q0, q1, q2, q3, outb):
        qcat = jnp.concatenate([q0[:], q1[:], q2[:], q3[:]], axis=0)
        eye = jnp.eye(128, dtype=jnp.float32)
        outb[:] = lax.dot_general(
            qcat, eye, (((0,), (0,)), ((), ())),
            preferred_element_type=jnp.float32)

    qspec = lambda k: pl.BlockSpec(
        (MEMORY_DIM, _RB),
        lambda i, k=k: (0, jnp.minimum(k * (_VP // _RB) + i, last_blk)))
    return pl.pallas_call(
        body,
        grid=grid,
        in_specs=[qspec(0), qspec(1), qspec(2), qspec(3)],
        out_specs=pl.BlockSpec((_RB, 128), lambda i: (i, 0)),
        out_shape=jax.ShapeDtypeStruct((_VP, 128), jnp.float32),
    )(memT, memT, memT, memT)


def _sc_gather_extract(mem128, src, dst, t, last_update):
    B = src.shape[0]
    b_per_w = B // _NW
    hb_rows = b_per_w // 2
    mesh = plsc.VectorSubcoreMesh(core_axis_name="c", subcore_axis_name="s")

    @functools.partial(
        pl.kernel,
        mesh=mesh,
        compiler_params=pltpu.CompilerParams(needs_layout_passes=False),
        out_type=jax.ShapeDtypeStruct((B, 128), jnp.float32),
        scratch_types=[
            pltpu.VMEM((b_per_w,), jnp.int32),
            pltpu.VMEM((b_per_w,), jnp.int32),
            pltpu.VMEM((b_per_w,), jnp.int32),
            pltpu.VMEM((b_per_w,), jnp.int32),
            pltpu.VMEM((b_per_w,), jnp.int32),
            pltpu.VMEM((b_per_w,), jnp.int32),
            pltpu.VMEM((hb_rows, 128), jnp.float32),
            pltpu.VMEM((hb_rows, 128), jnp.float32),
            pltpu.VMEM((hb_rows, 128), jnp.float32),
            pltpu.VMEM((b_per_w,), jnp.float32),
            pltpu.VMEM((b_per_w,), jnp.float32),
            pltpu.VMEM((b_per_w,), jnp.float32),
            pltpu.SemaphoreType.DMA,
        ],
    )
    def k(mem_hbm, src_hbm, dst_hbm, t_hbm, lu_hbm, feat,
          sidx, didx, sp, dp, scb, dcb, srows, drows, fbuf, slu, tv, dtv, sem):
        wid = lax.axis_index("s") * _NC + lax.axis_index("c")
        base = wid * b_per_w
        pltpu.sync_copy(src_hbm.at[pl.ds(base, b_per_w)], sidx)
        pltpu.sync_copy(dst_hbm.at[pl.ds(base, b_per_w)], didx)
        pltpu.sync_copy(t_hbm.at[pl.ds(base, b_per_w)], tv)
        for i in range(b_per_w // _LANES):
            s = pl.ds(i * _LANES, _LANES)
            r = sidx[s]
            sp[s] = jnp.bitwise_and(r, _VP - 1)
            scb[s] = lax.shift_right_logical(r, _QSH) * 32
            r2 = didx[s]
            dp[s] = jnp.bitwise_and(r2, _VP - 1)
            dcb[s] = lax.shift_right_logical(r2, _QSH) * 32
        lus = [pltpu.async_copy(lu_hbm.at[sidx.at[pl.ds(j * _CHUNK, _CHUNK)]],
                                slu.at[pl.ds(j * _CHUNK, _CHUNK)], sem)
               for j in range(b_per_w // _CHUNK)]
        for c in lus:
            c.wait()
        for i in range(b_per_w // _LANES):
            s = pl.ds(i * _LANES, _LANES)
            dtv[s] = tv[s] - slu[s]

        lane = lax.iota(jnp.int32, _LANES)
        for h in range(2):
            hb = h * hb_rows
            cs = []
            for j in range(hb_rows // _CHUNK):
                sl = pl.ds(hb + j * _CHUNK, _CHUNK)
                dsl = pl.ds(j * _CHUNK, _CHUNK)
                cs.append(pltpu.async_copy(mem_hbm.at[sp.at[sl]], srows.at[dsl], sem))
                cs.append(pltpu.async_copy(mem_hbm.at[dp.at[sl]], drows.at[dsl], sem))
            for c in cs:
                c.wait()

            def ebody(g, carry):
                rows = lane + g * _LANES
                scv = plsc.load_gather(scb, [rows + hb])
                dcv = plsc.load_gather(dcb, [rows + hb])
                dtl = plsc.load_gather(dtv, [rows + hb])
                for j in range(MEMORY_DIM):
                    v = plsc.load_gather(srows, [rows, scv + j])
                    plsc.store_scatter(fbuf, [rows, jnp.full((_LANES,), j, jnp.int32)], v)
                    v2 = plsc.load_gather(drows, [rows, dcv + j])
                    plsc.store_scatter(
                        fbuf, [rows, jnp.full((_LANES,), MEMORY_DIM + j, jnp.int32)], v2)
                plsc.store_scatter(
                    fbuf, [rows, jnp.full((_LANES,), 2 * MEMORY_DIM, jnp.int32)], dtl)
                return carry

            lax.fori_loop(0, hb_rows // _LANES, ebody, 0)
            pltpu.sync_copy(fbuf, feat.at[pl.ds(base + hb, hb_rows)])

    return k(mem128, src, dst, t, last_update)


def _mlp_body(fb, eat, wt, bt, w1a, w1b, w1c, w1d, b1r, w2t, b2r, outT):
    sm = fb[:, 0:MEMORY_DIM]
    dm = fb[:, MEMORY_DIM:2 * MEMORY_DIM]
    dtc = fb[:, 2 * MEMORY_DIM:2 * MEMORY_DIM + 1]
    enc = jnp.cos(dtc * wt[:] + bt[:])
    h = jnp.dot(sm, w1a[:], preferred_element_type=jnp.float32)
    h += jnp.dot(dm, w1b[:], preferred_element_type=jnp.float32)
    h += jnp.dot(enc, w1c[:], preferred_element_type=jnp.float32)
    h += lax.dot_general(eat[:], w1d[:], (((0,), (0,)), ((), ())),
                         preferred_element_type=jnp.float32)
    h = jnp.maximum(h + b1r[:], 0.0)
    outT[:] = lax.dot_general(w2t[:], h, (((1,), (1,)), ((), ())),
                              preferred_element_type=jnp.float32) + b2r[0, 0]


def _tc_mlp(feat, edge_attr_T, W_time, b_time, W1, b1, W2_T, b2):
    B = feat.shape[0]
    BLK = 2048
    grid = (B // BLK,)
    blk = lambda r, c: pl.BlockSpec((r, c), lambda i: (i, 0))
    full = lambda r, c: pl.BlockSpec((r, c), lambda i: (0, 0))
    outT = pl.pallas_call(
        _mlp_body,
        grid=grid,
        in_specs=[
            blk(BLK, 128),
            pl.BlockSpec((EDGE_FEAT_DIM, BLK), lambda i: (0, i)),
            full(1, TIME_DIM),
            full(1, TIME_DIM),
            full(MEMORY_DIM, HIDDEN),
            full(MEMORY_DIM, HIDDEN),
            full(TIME_DIM, HIDDEN),
            full(EDGE_FEAT_DIM, HIDDEN),
            full(1, HIDDEN),
            full(1, HIDDEN),
            full(1, 1),
        ],
        out_specs=pl.BlockSpec((1, BLK), lambda i: (0, i)),
        out_shape=jax.ShapeDtypeStruct((1, B), jnp.float32),
    )(feat, edge_attr_T, W_time, b_time,
      W1[0:MEMORY_DIM], W1[MEMORY_DIM:2 * MEMORY_DIM],
      W1[2 * MEMORY_DIM:2 * MEMORY_DIM + TIME_DIM],
      W1[2 * MEMORY_DIM + TIME_DIM:],
      b1.reshape(1, HIDDEN), W2_T, b2.reshape(1, 1))
    return outT.T


def kernel(src, dst, t, edge_attr, memory, last_update,
           W_time, b_time, W1, b1, W2, b2):
    mem128 = _tc_relayout(memory.T)
    feat = _sc_gather_extract(
        mem128, src.astype(jnp.int32), dst.astype(jnp.int32), t, last_update)
    return _tc_mlp(feat, edge_attr.astype(jnp.float32).T,
                   W_time, b_time.reshape(1, TIME_DIM), W1, b1, W2.T, b2)
